# baseline (speedup 1.0000x reference)
import jax
import jax.numpy as jnp
import numpy as np
from jax.experimental import pallas as pl
from jax.experimental.pallas import tpu as pltpu

_BS = 16
_A = 3
_C = 80
_ATTRS = 5 + _C
_H = 76
_W = 76
_HW = _H * _W
_STRIDE = 8.0
_ANCHORS = np.array([[116.0, 90.0], [156.0, 198.0], [373.0, 326.0]],
                    dtype=np.float32)

_B2 = 8
_I = 4
_S = _I * _W
_K = _H // _I
_WIN = 512
_FL = 384


def _decode_kernel(x_ref, o_ref, win, carry, head, tbuf, wsem, csem):
    b2 = pl.program_id(0)
    k = pl.program_id(1)
    bo = pl.multiple_of(b2 * _B2, _B2)

    n = jax.lax.broadcasted_iota(jnp.int32, (1, 1, _S), 2)
    gx = (n % _W).astype(jnp.float32)
    gy = (k * _I + n // _W).astype(jnp.float32)
    li = jax.lax.broadcasted_iota(jnp.int32, (1, 1, _WIN), 2)
    z208 = jnp.zeros((_ATTRS, _B2, _WIN - _S), jnp.float32)

    xm = x_ref[...].reshape(_S, _B2, _A * _ATTRS)
    half = _S // 2
    tbuf[:, :, 0:half] = jnp.transpose(xm[0:half], (2, 1, 0))
    tbuf[:, :, half:_S] = jnp.transpose(xm[half:_S], (2, 1, 0))

    for a in range(_A):
        v = tbuf[a * _ATTRS:(a + 1) * _ATTRS]

        s = jax.nn.sigmoid(v)
        row0 = (s[0:1] + gx) * _STRIDE
        row1 = (s[1:2] + gy) * _STRIDE
        row2 = jnp.exp(v[2:3]) * float(_ANCHORS[a, 0])
        row3 = jnp.exp(v[3:4]) * float(_ANCHORS[a, 1])
        r = jnp.concatenate([row0, row1, row2, row3, s[4:]], axis=0)

        if a >= 1:
            @pl.when(k == 0)
            def _():
                head[a - 1] = r[:, :, 0:128]

        g = a * _HW + k * _S
        d = jax.lax.rem(g, 128)
        w0 = pl.multiple_of(g - d, 128)

        rolled_r = pltpu.roll(jnp.concatenate([r, z208], axis=2), d, 2)
        prev_part = jnp.where(d >= 48,
                              carry[a, :, :, 0:128],
                              carry[a, :, :, 128:256])
        carry[a] = rolled_r[:, :, 256:512]
        li128 = jax.lax.broadcasted_iota(jnp.int32, (1, 1, 128), 2)

        @pl.when(k >= 1)
        def _():
            pltpu.make_async_copy(
                win.at[a], o_ref.at[:, pl.ds(bo, _B2), pl.ds(0, _FL)], wsem.at[a]).wait()

        win[a, :, :, 0:128] = jnp.where(
            li128 < d, prev_part, rolled_r[:, :, 0:128])
        win[a, :, :, 128:_FL] = rolled_r[:, :, 128:_FL]
        pltpu.make_async_copy(
            win.at[a], o_ref.at[:, pl.ds(bo, _B2), pl.ds(w0, _FL)], wsem.at[a]).start()

    @pl.when(k == _K - 1)
    def _():
        for a in range(_A):
            pltpu.make_async_copy(
                win.at[a], o_ref.at[:, pl.ds(bo, _B2), pl.ds(0, _FL)], wsem.at[a]).wait()
        li128 = jax.lax.broadcasted_iota(jnp.int32, (1, 1, 128), 2)
        for a in range(_A - 1):
            m = 16 * (a + 1)
            rt = carry[a, :, :, 128:256]
            hh = pltpu.roll(head[a], m, 2)
            win[a, :, :, 0:128] = jnp.where(li128 < m, rt, hh)
            e = pl.multiple_of((320 * (a + 1)) * k, 128)
            pltpu.make_async_copy(
                win.at[a, :, :, pl.ds(0, 128)],
                o_ref.at[:, pl.ds(bo, _B2), pl.ds(e, 128)], csem.at[a]).start()
        for a in range(_A - 1):
            pltpu.make_async_copy(
                win.at[a, :, :, pl.ds(0, 128)],
                o_ref.at[:, pl.ds(bo, _B2), pl.ds(0, 128)], csem.at[a]).wait()


def kernel(input):
    xt = jnp.transpose(input, (2, 3, 0, 1))
    out = pl.pallas_call(
        _decode_kernel,
        grid=(_BS // _B2, _K),
        in_specs=[
            pl.BlockSpec((_I, _H, _B2, _A * _ATTRS),
                         lambda b2, k: (k, 0, b2, 0)),
        ],
        compiler_params=pltpu.CompilerParams(
            dimension_semantics=("parallel", "arbitrary")),
        out_specs=pl.BlockSpec(memory_space=pltpu.HBM),
        out_shape=jax.ShapeDtypeStruct((_ATTRS, _BS, _A * _HW), jnp.float32),
        scratch_shapes=[
            pltpu.VMEM((_A, _ATTRS, _B2, _FL), jnp.float32),
            pltpu.VMEM((_A, _ATTRS, _B2, 256), jnp.float32),
            pltpu.VMEM((_A - 1, _ATTRS, _B2, 128), jnp.float32),
            pltpu.VMEM((_A * _ATTRS, _B2, _S), jnp.float32),
            pltpu.SemaphoreType.DMA((_A,)),
            pltpu.SemaphoreType.DMA((_A - 1,)),
        ],
    )(xt)
    return jnp.transpose(out, (1, 2, 0))


# speedup vs baseline: 2.1277x; 2.1277x over previous
import jax
import jax.numpy as jnp
import numpy as np
from jax.experimental import pallas as pl
from jax.experimental.pallas import tpu as pltpu

_BS = 16
_A = 3
_C = 80
_ATTRS = 5 + _C
_H = 76
_W = 76
_HW = _H * _W
_STRIDE = 8.0
_ANCHORS = np.array([[116.0, 90.0], [156.0, 198.0], [373.0, 326.0]],
                    dtype=np.float32)

_P = 512
_KC = 12
_LAST = _HW - (_KC - 1) * _P


def _decode_kernel(x_ref, o_ref, win, carry, head, resid, wsem, csem):
    k = pl.program_id(0)

    n = jax.lax.broadcasted_iota(jnp.int32, (1, 1, _P), 2) + k * _P
    gx = (n % _W).astype(jnp.float32)
    gy = (n // _W).astype(jnp.float32)

    for a in range(_A):
        da = 16 * a
        m = 16 * (a + 1)

        xa = x_ref[:, a]
        t = jnp.transpose(xa, (1, 0, 2))

        s = jax.nn.sigmoid(t)
        row0 = (s[0:1] + gx) * _STRIDE
        row1 = (s[1:2] + gy) * _STRIDE
        row2 = jnp.exp(t[2:3]) * float(_ANCHORS[a, 0])
        row3 = jnp.exp(t[3:4]) * float(_ANCHORS[a, 1])
        r = jnp.concatenate([row0, row1, row2, row3, s[4:]], axis=0)

        if a >= 1:
            @pl.when(k == 0)
            def _():
                head[a - 1] = r[:, :, 0:128]

        if a == 0:
            w = r
        else:
            w = jnp.concatenate(
                [carry[a - 1, :, :, 0:da], r[:, :, 0:_P - da]], axis=2)
            carry[a - 1, :, :, 0:da] = r[:, :, _P - da:_P]

        @pl.when(k == _KC - 1)
        def _():
            resid[a, :, :, 0:m] = r[:, :, _LAST - m:_LAST]

        @pl.when(jnp.logical_and(k >= 1, k <= _KC - 1))
        def _():
            pltpu.make_async_copy(
                win.at[a], o_ref.at[:, :, pl.ds(0, _P)], wsem.at[a]).wait()

        win[a] = w
        dst0 = pl.multiple_of(5760 * a + _P * k, 128)

        @pl.when(k < _KC - 1)
        def _():
            pltpu.make_async_copy(
                win.at[a], o_ref.at[:, :, pl.ds(dst0, _P)],
                wsem.at[a]).start()

        @pl.when(k == _KC - 1)
        def _():
            pltpu.make_async_copy(
                win.at[a, :, :, pl.ds(0, 128)],
                o_ref.at[:, :, pl.ds(dst0, 128)], wsem.at[a]).start()

    @pl.when(k == _KC - 1)
    def _():
        for a in range(_A):
            pltpu.make_async_copy(
                win.at[a, :, :, pl.ds(0, 128)],
                o_ref.at[:, :, pl.ds(0, 128)], wsem.at[a]).wait()
        for a in range(_A):
            m = 16 * (a + 1)
            if a < _A - 1:
                tail = head[a, :, :, 0:128 - m]
            else:
                tail = jnp.zeros((_ATTRS, _BS, 128 - m), jnp.float32)
            win[a, :, :, 0:128] = jnp.concatenate(
                [resid[a, :, :, 0:m], tail], axis=2)
            e = pl.multiple_of(5760 * a + _P * k + 128, 128)
            pltpu.make_async_copy(
                win.at[a, :, :, pl.ds(0, 128)],
                o_ref.at[:, :, pl.ds(e, 128)], csem.at[a]).start()
        for a in range(_A):
            pltpu.make_async_copy(
                win.at[a, :, :, pl.ds(0, 128)],
                o_ref.at[:, :, pl.ds(0, 128)], csem.at[a]).wait()


def kernel(input):
    x2 = input.reshape(_BS, _A, _ATTRS, _HW)
    out = pl.pallas_call(
        _decode_kernel,
        grid=(_KC,),
        in_specs=[
            pl.BlockSpec((_BS, _A, _ATTRS, _P), lambda k: (0, 0, 0, k)),
        ],
        out_specs=pl.BlockSpec(memory_space=pltpu.HBM),
        out_shape=jax.ShapeDtypeStruct((_ATTRS, _BS, _A * _HW), jnp.float32),
        scratch_shapes=[
            pltpu.VMEM((_A, _ATTRS, _BS, _P), jnp.float32),
            pltpu.VMEM((_A - 1, _ATTRS, _BS, 32), jnp.float32),
            pltpu.VMEM((_A - 1, _ATTRS, _BS, 128), jnp.float32),
            pltpu.VMEM((_A, _ATTRS, _BS, 48), jnp.float32),
            pltpu.SemaphoreType.DMA((_A,)),
            pltpu.SemaphoreType.DMA((_A,)),
        ],
    )(x2)
    return jnp.transpose(out, (1, 2, 0))


# speedup vs baseline: 2.1350x; 1.0034x over previous
import jax
import jax.numpy as jnp
import numpy as np
from jax.experimental import pallas as pl
from jax.experimental.pallas import tpu as pltpu

_BS = 16
_A = 3
_C = 80
_ATTRS = 5 + _C
_H = 76
_W = 76
_HW = _H * _W
_STRIDE = 8.0
_ANCHORS = np.array([[116.0, 90.0], [156.0, 198.0], [373.0, 326.0]],
                    dtype=np.float32)

_P = 768
_KC = 8
_LAST = _HW - (_KC - 1) * _P
_LFL = 384


def _decode_kernel(x_ref, o_ref, win, carry, head, resid, wsem, csem):
    k = pl.program_id(0)

    n = jax.lax.broadcasted_iota(jnp.int32, (1, 1, _P), 2) + k * _P
    gx = (n % _W).astype(jnp.float32)
    gy = (n // _W).astype(jnp.float32)

    for a in range(_A):
        da = 16 * a
        m = 16 * (a + 1)

        xa = x_ref[:, a]
        t = jnp.transpose(xa, (1, 0, 2))

        s = jax.nn.sigmoid(t)
        row0 = (s[0:1] + gx) * _STRIDE
        row1 = (s[1:2] + gy) * _STRIDE
        row2 = jnp.exp(t[2:3]) * float(_ANCHORS[a, 0])
        row3 = jnp.exp(t[3:4]) * float(_ANCHORS[a, 1])
        r = jnp.concatenate([row0, row1, row2, row3, s[4:]], axis=0)

        if a >= 1:
            @pl.when(k == 0)
            def _():
                head[a - 1] = r[:, :, 0:128]

        if a == 0:
            w = r
        else:
            w = jnp.concatenate(
                [carry[a - 1, :, :, 0:da], r[:, :, 0:_P - da]], axis=2)
            carry[a - 1, :, :, 0:da] = r[:, :, _P - da:_P]

        @pl.when(k == _KC - 1)
        def _():
            resid[a, :, :, 0:m] = r[:, :, _LAST - m:_LAST]

        @pl.when(jnp.logical_and(k >= 1, k <= _KC - 1))
        def _():
            pltpu.make_async_copy(
                win.at[a], o_ref.at[:, :, pl.ds(0, _P)], wsem.at[a]).wait()

        win[a] = w
        dst0 = pl.multiple_of(5760 * a + _P * k, 128)

        @pl.when(k < _KC - 1)
        def _():
            pltpu.make_async_copy(
                win.at[a], o_ref.at[:, :, pl.ds(dst0, _P)],
                wsem.at[a]).start()

        @pl.when(k == _KC - 1)
        def _():
            pltpu.make_async_copy(
                win.at[a, :, :, pl.ds(0, _LFL)],
                o_ref.at[:, :, pl.ds(dst0, _LFL)], wsem.at[a]).start()

    @pl.when(k == _KC - 1)
    def _():
        for a in range(_A):
            pltpu.make_async_copy(
                win.at[a, :, :, pl.ds(0, _LFL)],
                o_ref.at[:, :, pl.ds(0, _LFL)], wsem.at[a]).wait()
        for a in range(_A):
            m = 16 * (a + 1)
            if a < _A - 1:
                tail = head[a, :, :, 0:128 - m]
            else:
                tail = jnp.zeros((_ATTRS, _BS, 128 - m), jnp.float32)
            win[a, :, :, 0:128] = jnp.concatenate(
                [resid[a, :, :, 0:m], tail], axis=2)
            e = pl.multiple_of(5760 * a + _P * k + _LFL, 128)
            pltpu.make_async_copy(
                win.at[a, :, :, pl.ds(0, 128)],
                o_ref.at[:, :, pl.ds(e, 128)], csem.at[a]).start()
        for a in range(_A):
            pltpu.make_async_copy(
                win.at[a, :, :, pl.ds(0, 128)],
                o_ref.at[:, :, pl.ds(0, 128)], csem.at[a]).wait()


def kernel(input):
    x2 = input.reshape(_BS, _A, _ATTRS, _HW)
    out = pl.pallas_call(
        _decode_kernel,
        grid=(_KC,),
        in_specs=[
            pl.BlockSpec((_BS, _A, _ATTRS, _P), lambda k: (0, 0, 0, k)),
        ],
        out_specs=pl.BlockSpec(memory_space=pltpu.HBM),
        out_shape=jax.ShapeDtypeStruct((_ATTRS, _BS, _A * _HW), jnp.float32),
        scratch_shapes=[
            pltpu.VMEM((_A, _ATTRS, _BS, _P), jnp.float32),
            pltpu.VMEM((_A - 1, _ATTRS, _BS, 32), jnp.float32),
            pltpu.VMEM((_A - 1, _ATTRS, _BS, 128), jnp.float32),
            pltpu.VMEM((_A, _ATTRS, _BS, 48), jnp.float32),
            pltpu.SemaphoreType.DMA((_A,)),
            pltpu.SemaphoreType.DMA((_A,)),
        ],
    )(x2)
    return jnp.transpose(out, (1, 2, 0))
